# Initial kernel scaffold; baseline (speedup 1.0000x reference)
import jax
import jax.numpy as jnp
from jax.experimental import pallas as pl


def kernel(pos, params, batch):
    raise NotImplementedError("write your pallas kernel here")


# speedup vs baseline: 3.2388x; 3.2388x over previous
import functools

import jax
import jax.numpy as jnp
from jax import lax
from jax.experimental import pallas as pl
from jax.experimental.pallas import tpu as pltpu
from jax.experimental.pallas import tpu_sc as plsc

K = 20
NUM_GRAPHS = 16
EPS = 1e-5
SLOPE = 0.2


def _knn_call(pos, batch, N, BQ=8, W=4096):
    W = min(W, N)
    nchunk = N // BQ

    def body(ws_ref, q_ref, qb_ref, post_ref, batchr_ref, out_ref):
        c = pl.program_id(0)
        start = pl.multiple_of(ws_ref[c], 128)
        q = q_ref[...]
        qb = qb_ref[...]
        pwin = post_ref[:, pl.ds(start, W)]
        pb = batchr_ref[:, pl.ds(start, W)]

        d = jnp.zeros((BQ, W), jnp.float32)
        for dim in range(3):
            diff = q[:, dim:dim + 1] - pwin[dim:dim + 1, :]
            d = d + diff * diff

        rg = c * BQ + lax.broadcasted_iota(jnp.int32, (BQ, 1), 0)
        cg = start + lax.broadcasted_iota(jnp.int32, (1, W), 1)
        bad = (qb != pb) | (rg == cg)
        d = jnp.where(bad, jnp.inf, d)

        idxw = lax.broadcasted_iota(jnp.int32, (BQ, W), 1)
        cols = []
        for _ in range(K):
            m = jnp.min(d, axis=1, keepdims=True)
            am = jnp.min(jnp.where(d <= m, idxw, N), axis=1, keepdims=True)
            cols.append(am + start)
            d = jnp.where(idxw == am, jnp.inf, d)
        out_ref[...] = jnp.concatenate(cols, axis=1)

    starts = jnp.searchsorted(batch, batch[0::BQ]).astype(jnp.int32)
    ws = jnp.minimum((starts // 128) * 128, N - W)

    grid_spec = pltpu.PrefetchScalarGridSpec(
        num_scalar_prefetch=1,
        grid=(nchunk,),
        in_specs=[
            pl.BlockSpec((BQ, 3), lambda c, ws: (c, 0)),
            pl.BlockSpec((BQ, 1), lambda c, ws: (c, 0)),
            pl.BlockSpec((3, N), lambda c, ws: (0, 0)),
            pl.BlockSpec((1, N), lambda c, ws: (0, 0)),
        ],
        out_specs=pl.BlockSpec((BQ, K), lambda c, ws: (c, 0)),
    )
    return pl.pallas_call(
        body,
        grid_spec=grid_spec,
        out_shape=jax.ShapeDtypeStruct((N, K), jnp.int32),
    )(ws, pos, batch[:, None], pos.T, batch[None, :])


def _gather_rows(src, v, E, Fg, CB=128):
    NW = 32
    per_w = E // NW
    n_it = per_w // CB
    mesh = plsc.VectorSubcoreMesh(core_axis_name="c", subcore_axis_name="s")

    @functools.partial(
        pl.kernel,
        out_type=jax.ShapeDtypeStruct((E, Fg), jnp.float32),
        mesh=mesh,
        scratch_types=[
            pltpu.VMEM((CB,), jnp.int32),
            pltpu.VMEM((CB, Fg), jnp.float32),
            pltpu.SemaphoreType.DMA,
        ],
    )
    def gk(src_hbm, v_hbm, out_hbm, idx_v, rows_v, sem):
        wid = lax.axis_index("s") * 2 + lax.axis_index("c")
        base = wid * per_w

        def it(t, carry):
            off = base + t * CB
            pltpu.sync_copy(src_hbm.at[pl.ds(off, CB)], idx_v)
            pltpu.async_copy(v_hbm.at[idx_v], rows_v, sem).wait()
            pltpu.sync_copy(rows_v, out_hbm.at[pl.ds(off, CB)])
            return carry

        lax.fori_loop(0, n_it, it, 0)

    return gk(src, v)


def _node_prep(xm, stats, g, be, N, E, BA=512, FP=128):
    Fi = xm.shape[1]
    nblk = N // BA
    Ef = float(E)

    def body(m_ref, st_ref, g_ref, be_ref, xp_ref):
        s = st_ref[0:1, :]
        q = st_ref[1:2, :]
        mu = s / Ef
        var = q / Ef - mu * mu
        x = (m_ref[...] - mu) / jnp.sqrt(var + EPS) * g_ref[...] + be_ref[...]
        x = jnp.where(x >= 0, x, SLOPE * x)
        if Fi < FP:
            x = jnp.concatenate([x, jnp.zeros((BA, FP - Fi), jnp.float32)],
                                axis=1)
        xp_ref[...] = x

    return pl.pallas_call(
        body,
        grid=(nblk,),
        in_specs=[
            pl.BlockSpec((BA, Fi), lambda i: (i, 0)),
            pl.BlockSpec((2, Fi), lambda i: (0, 0)),
            pl.BlockSpec((1, Fi), lambda i: (0, 0)),
            pl.BlockSpec((1, Fi), lambda i: (0, 0)),
        ],
        out_specs=pl.BlockSpec((BA, FP), lambda i: (i, 0)),
        out_shape=jax.ShapeDtypeStruct((N, FP), jnp.float32),
    )(xm, stats, g, be)


def _edge_stats(xpad, xg, w1T, b1, N, Fi, Fo, BC=256, FP=128):
    nblk = N // BC

    def body(xp_ref, xg_ref, w1_ref, b1_ref, out_ref):
        pid = pl.program_id(0)
        xi = xp_ref[...][:, :Fi]
        xg3 = xg_ref[...].reshape(BC, K, FP)
        w1 = w1_ref[...]
        b1v = b1_ref[...]
        s = jnp.zeros((1, Fo), jnp.float32)
        q = jnp.zeros((1, Fo), jnp.float32)
        for k in range(K):
            e = jnp.concatenate([xi, xg3[:, k, :Fi] - xi], axis=1)
            y = jnp.dot(e, w1, preferred_element_type=jnp.float32) + b1v
            s = s + jnp.sum(y, axis=0, keepdims=True)
            q = q + jnp.sum(y * y, axis=0, keepdims=True)

        @pl.when(pid == 0)
        def _():
            out_ref[...] = jnp.zeros_like(out_ref)

        out_ref[0:1, :] += s
        out_ref[1:2, :] += q

    return pl.pallas_call(
        body,
        grid=(nblk,),
        in_specs=[
            pl.BlockSpec((BC, FP), lambda i: (i, 0)),
            pl.BlockSpec((BC * K, FP), lambda i: (i, 0)),
            pl.BlockSpec((2 * Fi, Fo), lambda i: (0, 0)),
            pl.BlockSpec((1, Fo), lambda i: (0, 0)),
        ],
        out_specs=pl.BlockSpec((2, Fo), lambda i: (0, 0)),
        out_shape=jax.ShapeDtypeStruct((2, Fo), jnp.float32),
    )(xpad, xg, w1T, b1)


def _edge_conv(xpad, xg, st1, g1, be1, w1T, b1, w2T, b2, N, Fi, Fo, E,
               BD=256, FP=128):
    nblk = N // BD
    Ef = float(E)

    def body(xp_ref, xg_ref, st_ref, g1_ref, be1_ref, w1_ref, b1_ref,
             w2_ref, b2_ref, m_ref, st2_ref):
        pid = pl.program_id(0)
        s = st_ref[0:1, :]
        q = st_ref[1:2, :]
        mu = s / Ef
        var = q / Ef - mu * mu
        sd = jnp.sqrt(var + EPS)
        g1v = g1_ref[...]
        be = be1_ref[...]
        xi = xp_ref[...][:, :Fi]
        xg3 = xg_ref[...].reshape(BD, K, FP)
        w1 = w1_ref[...]
        b1v = b1_ref[...]
        w2 = w2_ref[...]
        b2v = b2_ref[...]
        mx = jnp.full((BD, Fo), -jnp.inf, jnp.float32)
        s2 = jnp.zeros((1, Fo), jnp.float32)
        q2 = jnp.zeros((1, Fo), jnp.float32)
        for k in range(K):
            e = jnp.concatenate([xi, xg3[:, k, :Fi] - xi], axis=1)
            y1 = jnp.dot(e, w1, preferred_element_type=jnp.float32) + b1v
            z = (y1 - mu) / sd * g1v + be
            z = jnp.where(z >= 0, z, SLOPE * z)
            y2 = jnp.dot(z, w2, preferred_element_type=jnp.float32) + b2v
            mx = jnp.maximum(mx, y2)
            s2 = s2 + jnp.sum(y2, axis=0, keepdims=True)
            q2 = q2 + jnp.sum(y2 * y2, axis=0, keepdims=True)
        m_ref[...] = mx

        @pl.when(pid == 0)
        def _():
            st2_ref[...] = jnp.zeros_like(st2_ref)

        st2_ref[0:1, :] += s2
        st2_ref[1:2, :] += q2

    return pl.pallas_call(
        body,
        grid=(nblk,),
        in_specs=[
            pl.BlockSpec((BD, FP), lambda i: (i, 0)),
            pl.BlockSpec((BD * K, FP), lambda i: (i, 0)),
            pl.BlockSpec((2, Fo), lambda i: (0, 0)),
            pl.BlockSpec((1, Fo), lambda i: (0, 0)),
            pl.BlockSpec((1, Fo), lambda i: (0, 0)),
            pl.BlockSpec((2 * Fi, Fo), lambda i: (0, 0)),
            pl.BlockSpec((1, Fo), lambda i: (0, 0)),
            pl.BlockSpec((Fo, Fo), lambda i: (0, 0)),
            pl.BlockSpec((1, Fo), lambda i: (0, 0)),
        ],
        out_specs=[
            pl.BlockSpec((BD, Fo), lambda i: (i, 0)),
            pl.BlockSpec((2, Fo), lambda i: (0, 0)),
        ],
        out_shape=[
            jax.ShapeDtypeStruct((N, Fo), jnp.float32),
            jax.ShapeDtypeStruct((2, Fo), jnp.float32),
        ],
    )(xpad, xg, st1, g1, be1, w1T, b1, w2T, b2)


def _head_l1(ms, sts, gs, bes, l1wT, l1b, N, E, BH=256):
    nblk = N // BH
    Fs = [m.shape[1] for m in ms]
    EMB = l1wT.shape[1]

    def body(*refs):
        m_refs = refs[0:4]
        st_refs = refs[4:8]
        g_refs = refs[8:12]
        be_refs = refs[12:16]
        w_ref, b_ref, y_ref, sty_ref = refs[16:]
        pid = pl.program_id(0)
        Ef = float(E)
        xs = []
        for i in range(4):
            s = st_refs[i][0:1, :]
            q = st_refs[i][1:2, :]
            mu = s / Ef
            var = q / Ef - mu * mu
            x = ((m_refs[i][...] - mu) / jnp.sqrt(var + EPS) * g_refs[i][...]
                 + be_refs[i][...])
            xs.append(jnp.where(x >= 0, x, SLOPE * x))
        xc = jnp.concatenate(xs, axis=1)
        y = jnp.dot(xc, w_ref[...],
                    preferred_element_type=jnp.float32) + b_ref[...]
        y_ref[...] = y

        @pl.when(pid == 0)
        def _():
            sty_ref[...] = jnp.zeros_like(sty_ref)

        sty_ref[0:1, :] += jnp.sum(y, axis=0, keepdims=True)
        sty_ref[1:2, :] += jnp.sum(y * y, axis=0, keepdims=True)

    in_specs = (
        [pl.BlockSpec((BH, F), lambda i: (i, 0)) for F in Fs]
        + [pl.BlockSpec((2, F), lambda i: (0, 0)) for F in Fs]
        + [pl.BlockSpec((1, F), lambda i: (0, 0)) for F in Fs]
        + [pl.BlockSpec((1, F), lambda i: (0, 0)) for F in Fs]
        + [pl.BlockSpec((512, EMB), lambda i: (0, 0)),
           pl.BlockSpec((1, EMB), lambda i: (0, 0))]
    )
    return pl.pallas_call(
        body,
        grid=(nblk,),
        in_specs=in_specs,
        out_specs=[
            pl.BlockSpec((BH, EMB), lambda i: (i, 0)),
            pl.BlockSpec((2, EMB), lambda i: (0, 0)),
        ],
        out_shape=[
            jax.ShapeDtypeStruct((N, EMB), jnp.float32),
            jax.ShapeDtypeStruct((2, EMB), jnp.float32),
        ],
    )(*ms, *sts, *gs, *bes, l1wT, l1b)


def _head_pool(y, sty, bn1g, bn1b, batch_col, N, BH=512):
    nblk = N // BH
    EMB = y.shape[1]

    def body(y_ref, st_ref, g_ref, be_ref, b_ref, gmax_ref, gsum_ref,
             gcnt_ref):
        pid = pl.program_id(0)
        s = st_ref[0:1, :]
        q = st_ref[1:2, :]
        mu = s / float(N)
        var = q / float(N) - mu * mu
        x = (y_ref[...] - mu) / jnp.sqrt(var + EPS) * g_ref[...] + be_ref[...]
        x = jnp.where(x >= 0, x, SLOPE * x)
        bcol = b_ref[...]

        @pl.when(pid == 0)
        def _():
            gmax_ref[...] = jnp.full_like(gmax_ref, -jnp.inf)
            gsum_ref[...] = jnp.zeros_like(gsum_ref)
            gcnt_ref[...] = jnp.zeros_like(gcnt_ref)

        bmin = b_ref[0, 0]
        bmax = b_ref[BH - 1, 0]
        for g in range(NUM_GRAPHS):
            @pl.when((g >= bmin) & (g <= bmax))
            def _(g=g):
                mask = bcol == g
                xm = jnp.max(jnp.where(mask, x, -jnp.inf), axis=0,
                             keepdims=True)
                xs = jnp.sum(jnp.where(mask, x, 0.0), axis=0, keepdims=True)
                cn = jnp.sum(jnp.where(mask, 1.0, 0.0))
                gmax_ref[g:g + 1, :] = jnp.maximum(gmax_ref[g:g + 1, :], xm)
                gsum_ref[g:g + 1, :] += xs
                gcnt_ref[g:g + 1, :] += jnp.zeros((1, 128), jnp.float32) + cn

    return pl.pallas_call(
        body,
        grid=(nblk,),
        in_specs=[
            pl.BlockSpec((BH, EMB), lambda i: (i, 0)),
            pl.BlockSpec((2, EMB), lambda i: (0, 0)),
            pl.BlockSpec((1, EMB), lambda i: (0, 0)),
            pl.BlockSpec((1, EMB), lambda i: (0, 0)),
            pl.BlockSpec((BH, 1), lambda i: (i, 0)),
        ],
        out_specs=[
            pl.BlockSpec((NUM_GRAPHS, EMB), lambda i: (0, 0)),
            pl.BlockSpec((NUM_GRAPHS, EMB), lambda i: (0, 0)),
            pl.BlockSpec((NUM_GRAPHS, 128), lambda i: (0, 0)),
        ],
        out_shape=[
            jax.ShapeDtypeStruct((NUM_GRAPHS, EMB), jnp.float32),
            jax.ShapeDtypeStruct((NUM_GRAPHS, EMB), jnp.float32),
            jax.ShapeDtypeStruct((NUM_GRAPHS, 128), jnp.float32),
        ],
    )(y, sty, bn1g, bn1b, batch_col)


def _head_mlp(gmax, gsum, gcnt, p):
    l2wT = p['l2_w'].T
    l3wT = p['l3_w'].T
    l4wT = p['l4_w'].T
    OUT = l4wT.shape[1]

    def bn_rows(x, g, b):
        mu = jnp.mean(x, axis=0, keepdims=True)
        var = jnp.mean((x - mu) * (x - mu), axis=0, keepdims=True)
        return (x - mu) / jnp.sqrt(var + EPS) * g + b

    def body(gmax_ref, gsum_ref, gcnt_ref, w2_ref, b2_ref, g2_ref, be2_ref,
             w3_ref, b3_ref, g3_ref, be3_ref, w4_ref, b4_ref, out_ref):
        cnt = gcnt_ref[:, 0:1]
        xmean = gsum_ref[...] / jnp.maximum(cnt, 1.0)
        x = jnp.concatenate([gmax_ref[...], xmean], axis=1)
        y = jnp.dot(x, w2_ref[...],
                    preferred_element_type=jnp.float32) + b2_ref[...]
        y = bn_rows(y, g2_ref[...], be2_ref[...])
        y = jnp.where(y >= 0, y, SLOPE * y)
        y = jnp.dot(y, w3_ref[...],
                    preferred_element_type=jnp.float32) + b3_ref[...]
        y = bn_rows(y, g3_ref[...], be3_ref[...])
        y = jnp.where(y >= 0, y, SLOPE * y)
        out_ref[...] = jnp.dot(y, w4_ref[...],
                               preferred_element_type=jnp.float32) + b4_ref[...]

    return pl.pallas_call(
        body,
        out_shape=jax.ShapeDtypeStruct((NUM_GRAPHS, OUT), jnp.float32),
    )(gmax, gsum, gcnt, l2wT, p['l2_b'][None, :], p['bn2_g'][None, :],
      p['bn2_b'][None, :], l3wT, p['l3_b'][None, :], p['bn3_g'][None, :],
      p['bn3_b'][None, :], l4wT, p['l4_b'][None, :])


def kernel(pos, params, batch):
    N = pos.shape[0]
    E = N * K
    p = params

    knn_idx = _knn_call(pos, batch, N)
    src = knn_idx.reshape(E)

    xm = pos
    stats = jnp.zeros((2, 3), jnp.float32)
    g_prev = jnp.ones((1, 3), jnp.float32)
    be_prev = jnp.zeros((1, 3), jnp.float32)
    FP = 128
    ms, sts, gs, bes = [], [], [], []
    for li, pfx in enumerate(('c1', 'c2', 'c3', 'c4')):
        w1 = p[pfx + '_w1']
        Fi = w1.shape[1] // 2
        Fo = w1.shape[0]
        w1T = w1.T
        if li == 0:
            xpad = jnp.pad(xm, ((0, 0), (0, FP - Fi)))
        else:
            xpad = _node_prep(xm, stats, g_prev, be_prev, N, E)
        xg = _gather_rows(src, xpad, E, FP)
        st1 = _edge_stats(xpad, xg, w1T, p[pfx + '_b1'][None, :], N, Fi, Fo)
        xm, stats = _edge_conv(xpad, xg, st1, p[pfx + '_g1'][None, :],
                               p[pfx + '_be1'][None, :], w1T,
                               p[pfx + '_b1'][None, :],
                               p[pfx + '_w2'].T, p[pfx + '_b2'][None, :],
                               N, Fi, Fo, E)
        g_prev = p[pfx + '_g2'][None, :]
        be_prev = p[pfx + '_be2'][None, :]
        ms.append(xm)
        sts.append(stats)
        gs.append(g_prev)
        bes.append(be_prev)

    y, sty = _head_l1(ms, sts, gs, bes, p['l1_w'].T, p['l1_b'][None, :], N, E)
    gmax, gsum, gcnt = _head_pool(y, sty, p['bn1_g'][None, :],
                                  p['bn1_b'][None, :],
                                  batch[:, None].astype(jnp.int32), N)
    return _head_mlp(gmax, gsum, gcnt, params)
